# Initial kernel scaffold; baseline (speedup 1.0000x reference)
import jax
import jax.numpy as jnp
from jax.experimental import pallas as pl


def kernel(x0, params):
    raise NotImplementedError("write your pallas kernel here")


# speedup vs baseline: 4.3157x; 4.3157x over previous
import functools
import math

import jax
import jax.numpy as jnp
from jax import lax
from jax.experimental import pallas as pl
from jax.experimental.pallas import tpu as pltpu
from jax.experimental.pallas import tpu_sc as plsc

_EPS = 1e-5
_K = 32


def _knn_body(rows_ref, cols_ref, a_ref, d_ref, bias_ref,
              idx_ref, g_ref, h_ref, *, n, blk, k):
    b = pl.program_id(0)
    rows = rows_ref[0]
    cols = cols_ref[0]
    inner = -2.0 * jnp.dot(rows, cols, preferred_element_type=jnp.float32)
    xx_r = jnp.sum(rows * rows, axis=1, keepdims=True)
    xx_c = jnp.sum(cols * cols, axis=0, keepdims=True)
    vals = (xx_c + inner) + xx_r
    iota = lax.broadcasted_iota(jnp.int32, (blk, n), 1)
    outs = []
    for _ in range(k):
        m = jnp.min(vals, axis=1, keepdims=True)
        am = jnp.min(jnp.where(vals == m, iota, n), axis=1, keepdims=True)
        vals = jnp.where(iota == am, jnp.inf, vals)
        outs.append(am)
    idx_ref[0] = jnp.concatenate(outs, axis=1) + b * n
    g_ref[0] = jnp.dot(rows, a_ref[...], preferred_element_type=jnp.float32)
    h_ref[0] = (jnp.dot(rows, d_ref[...], preferred_element_type=jnp.float32)
                + bias_ref[...])


def _knn_call(rows, cols, a, d, bias, *, k, blk):
    bsz, n, c = rows.shape
    grid = (bsz, n // blk)
    return pl.pallas_call(
        functools.partial(_knn_body, n=n, blk=blk, k=k),
        grid=grid,
        in_specs=[
            pl.BlockSpec((1, blk, c), lambda b, i: (b, i, 0)),
            pl.BlockSpec((1, c, n), lambda b, i: (b, 0, 0)),
            pl.BlockSpec((c, 64), lambda b, i: (0, 0)),
            pl.BlockSpec((c, 64), lambda b, i: (0, 0)),
            pl.BlockSpec((1, 64), lambda b, i: (0, 0)),
        ],
        out_specs=[
            pl.BlockSpec((1, blk, k), lambda b, i: (b, i, 0)),
            pl.BlockSpec((1, blk, 64), lambda b, i: (b, i, 0)),
            pl.BlockSpec((1, blk, 64), lambda b, i: (b, i, 0)),
        ],
        out_shape=[
            jax.ShapeDtypeStruct((bsz, n, k), jnp.int32),
            jax.ShapeDtypeStruct((bsz, n, 64), jnp.float32),
            jax.ShapeDtypeStruct((bsz, n, 64), jnp.float32),
        ],
    )(rows, cols, a, d, bias)


def _sc_gather(table, idx):
    info = plsc.get_sparse_core_info()
    nw = info.num_cores * info.num_subcores
    m, dch = idx.shape[0], table.shape[1]
    per_w = m // nw
    ch = 1024
    nch = per_w // ch
    mesh = plsc.VectorSubcoreMesh(core_axis_name="c", subcore_axis_name="s")

    @functools.partial(
        pl.kernel, mesh=mesh,
        compiler_params=pltpu.CompilerParams(use_tc_tiling_on_sc=False),
        out_type=jax.ShapeDtypeStruct((m, dch), jnp.float32),
        scratch_types=[
            pltpu.VMEM((ch,), jnp.int32),
            pltpu.VMEM((ch, dch), jnp.float32),
            pltpu.SemaphoreType.DMA,
        ],
    )
    def gk(table_hbm, idx_hbm, out_hbm, idx_v, rows_v, sem):
        wid = lax.axis_index("s") * info.num_cores + lax.axis_index("c")

        def body(i, carry):
            base = wid * per_w + i * ch
            pltpu.sync_copy(idx_hbm.at[pl.ds(base, ch)], idx_v)
            pltpu.async_copy(table_hbm.at[idx_v], rows_v, sem).wait()
            pltpu.sync_copy(rows_v, out_hbm.at[pl.ds(base, ch)])
            return carry

        lax.fori_loop(0, nch, body, 0)

    return gk(table, idx)


def _shift_sum(zs, k):
    acc = zs[0]
    p = zs[0].shape[0]
    for dd in range(1, len(zs)):
        z = zs[dd]
        shifted = jnp.concatenate(
            [z[:, dd:, :], jnp.zeros((p, dd, 64), jnp.float32)], axis=1)
        acc = acc + shifted
    return acc


def _block_body(g_ref, h_ref, w2_ref, b2_ref, w3_ref, b3_ref, w4_ref, b4_ref,
                out_ref, *, p, k):
    g = g_ref[0].reshape(p, k, 64)
    h = h_ref[0]
    y = jnp.maximum(g + h[:, None, :], 0.0)
    for w_ref, b_ref, taps in ((w2_ref, b2_ref, 2),
                               (w3_ref, b3_ref, 4),
                               (w4_ref, b4_ref, 8)):
        flat = y.reshape(p * k, 64)
        zs = [jnp.dot(flat, w_ref[dd], preferred_element_type=jnp.float32)
                 .reshape(p, k, 64) for dd in range(taps)]
        y = jnp.maximum(_shift_sum(zs, k) + b_ref[...][None, :, :], 0.0)
    out_ref[0] = jnp.max(y[:, : k - 11, :], axis=1)


def _block_call(gath, h, w2, b2, w3, b3, w4, b4, *, k, p):
    bsz, n, _ = h.shape
    grid = (bsz, n // p)
    return pl.pallas_call(
        functools.partial(_block_body, p=p, k=k),
        grid=grid,
        in_specs=[
            pl.BlockSpec((1, p * k, 64), lambda b, i: (b, i, 0)),
            pl.BlockSpec((1, p, 64), lambda b, i: (b, i, 0)),
            pl.BlockSpec((2, 64, 64), lambda b, i: (0, 0, 0)),
            pl.BlockSpec((1, 64), lambda b, i: (0, 0)),
            pl.BlockSpec((4, 64, 64), lambda b, i: (0, 0, 0)),
            pl.BlockSpec((1, 64), lambda b, i: (0, 0)),
            pl.BlockSpec((8, 64, 64), lambda b, i: (0, 0, 0)),
            pl.BlockSpec((1, 64), lambda b, i: (0, 0)),
        ],
        out_specs=pl.BlockSpec((1, p, 64), lambda b, i: (b, i, 0)),
        out_shape=jax.ShapeDtypeStruct((bsz, n, 64), jnp.float32),
    )(gath, h, w2, b2, w3, b3, w4, b4)


def _head_body(x1_ref, x2_ref, p1a_ref, p1b_ref, q1_ref, p2_ref, q2_ref,
               p3_ref, q3_ref, out_ref, *, n):
    z = jnp.maximum(
        jnp.dot(x1_ref[0], p1a_ref[...], preferred_element_type=jnp.float32)
        + jnp.dot(x2_ref[0], p1b_ref[...], preferred_element_type=jnp.float32)
        + q1_ref[...], 0.0)
    z = jnp.maximum(
        jnp.dot(z, p2_ref[...], preferred_element_type=jnp.float32)
        + q2_ref[...], 0.0)
    z = jnp.maximum(
        jnp.dot(z, p3_ref[...], preferred_element_type=jnp.float32)
        + q3_ref[...], 0.0)
    out_ref[0] = jnp.sum(z, axis=0, keepdims=True) * (1.0 / n)


def _head_call(x1, x2, p1a, p1b, q1, p2, q2, p3, q3):
    bsz, n, _ = x1.shape
    no = p3.shape[1]
    return pl.pallas_call(
        functools.partial(_head_body, n=n),
        grid=(bsz,),
        in_specs=[
            pl.BlockSpec((1, n, 64), lambda b: (b, 0, 0)),
            pl.BlockSpec((1, n, 64), lambda b: (b, 0, 0)),
            pl.BlockSpec((64, 64), lambda b: (0, 0)),
            pl.BlockSpec((64, 64), lambda b: (0, 0)),
            pl.BlockSpec((1, 64), lambda b: (0, 0)),
            pl.BlockSpec((64, 256), lambda b: (0, 0)),
            pl.BlockSpec((1, 256), lambda b: (0, 0)),
            pl.BlockSpec((256, no), lambda b: (0, 0)),
            pl.BlockSpec((1, no), lambda b: (0, 0)),
        ],
        out_specs=pl.BlockSpec((1, 1, no), lambda b: (b, 0, 0)),
        out_shape=jax.ShapeDtypeStruct((bsz, 1, no), jnp.float32),
    )(x1, x2, p1a, p1b, q1, p2, q2, p3, q3)


def _fold_block(p, pref, cin):
    s = 1.0 / math.sqrt(1.0 + _EPS)
    s0 = p[pref + "_g0"] * s
    t0 = p[pref + "_b0"]
    w1 = p[pref + "_w1"][:, :, 0, 0]
    s1 = p[pref + "_g1"] * s
    w1eff = w1 * s1[:, None] * s0[None, :]
    bias1 = s1 * (w1 @ t0) + p[pref + "_b1"]
    half = cin // 2
    a = jnp.transpose(w1eff[:, :half])
    d = jnp.transpose(w1eff[:, half:] - w1eff[:, :half])
    ws, bs = [], []
    for i in (2, 3, 4):
        wi = p[pref + "_w%d" % i][:, :, 0, :]
        si = p[pref + "_g%d" % i] * s
        ws.append(jnp.transpose(wi * si[:, None, None], (2, 1, 0)))
        bs.append(p[pref + "_b%d" % i].reshape(1, 64))
    return a, d, bias1.reshape(1, 64), ws, bs


def _fold_head(p, nm):
    s = 1.0 / math.sqrt(1.0 + _EPS)
    w = p["wc" + nm][:, :, 0]
    sc = p["gc" + nm] * s
    wt = jnp.transpose(w * sc[:, None])
    b = (p["bc" + nm] * sc + p["bec" + nm]).reshape(1, -1)
    return wt, b


def kernel(x0, params):
    bsz, _, n = x0.shape
    blk = 256
    p_tile = 128

    a1, d1, bias1, ws1, bs1 = _fold_block(params, "c1", 6)
    a2, d2, bias2, ws2, bs2 = _fold_block(params, "c2", 128)
    p1, q1 = _fold_head(params, "1")
    p2, q2 = _fold_head(params, "2")
    p3, q3 = _fold_head(params, "3")

    xt0 = jnp.transpose(x0, (0, 2, 1))
    idx1, g1, h1 = _knn_call(xt0, x0, a1, d1, bias1, k=_K, blk=blk)
    gath1 = _sc_gather(g1.reshape(bsz * n, 64), idx1.reshape(-1))
    x1 = _block_call(gath1.reshape(bsz, n * _K, 64), h1,
                     ws1[0], bs1[0], ws1[1], bs1[1], ws1[2], bs1[2],
                     k=_K, p=p_tile)

    x1t = jnp.transpose(x1, (0, 2, 1))
    idx2, g2, h2 = _knn_call(x1, x1t, a2, d2, bias2, k=_K, blk=blk)
    gath2 = _sc_gather(g2.reshape(bsz * n, 64), idx2.reshape(-1))
    x2 = _block_call(gath2.reshape(bsz, n * _K, 64), h2,
                     ws2[0], bs2[0], ws2[1], bs2[1], ws2[2], bs2[2],
                     k=_K, p=p_tile)

    out = _head_call(x1, x2, p1[:64], p1[64:], q1, p2, q2, p3, q3)
    return out.reshape(bsz, p3.shape[1])


# speedup vs baseline: 4.6330x; 1.0735x over previous
import functools
import math

import jax
import jax.numpy as jnp
from jax import lax
from jax.experimental import pallas as pl
from jax.experimental.pallas import tpu as pltpu
from jax.experimental.pallas import tpu_sc as plsc

_EPS = 1e-5
_K = 32


def _knn_body(rows_ref, cols_ref, a_ref, d_ref, bias_ref,
              idx_ref, g_ref, h_ref, *, n, blk, k):
    b = pl.program_id(0)
    rows = rows_ref[0]
    cols = cols_ref[0]
    inner = -2.0 * jnp.dot(rows, cols, preferred_element_type=jnp.float32)
    xx_r = jnp.sum(rows * rows, axis=1, keepdims=True)
    xx_c = jnp.sum(cols * cols, axis=0, keepdims=True)
    vals = (xx_c + inner) + xx_r
    iota = lax.broadcasted_iota(jnp.int32, (blk, n), 1)
    outs = []
    for _ in range(k):
        m = jnp.min(vals, axis=1, keepdims=True)
        eq = vals == m
        am = jnp.min(jnp.where(eq, iota, n), axis=1, keepdims=True)
        vals = jnp.where(eq, jnp.inf, vals)
        outs.append(am)
    idx_ref[0] = jnp.concatenate(outs, axis=1) + b * n
    g_ref[0] = jnp.dot(rows, a_ref[...], preferred_element_type=jnp.float32)
    h_ref[0] = (jnp.dot(rows, d_ref[...], preferred_element_type=jnp.float32)
                + bias_ref[...])


def _knn_call(rows, cols, a, d, bias, *, k, blk):
    bsz, n, c = rows.shape
    grid = (bsz, n // blk)
    return pl.pallas_call(
        functools.partial(_knn_body, n=n, blk=blk, k=k),
        grid=grid,
        in_specs=[
            pl.BlockSpec((1, blk, c), lambda b, i: (b, i, 0)),
            pl.BlockSpec((1, c, n), lambda b, i: (b, 0, 0)),
            pl.BlockSpec((c, 64), lambda b, i: (0, 0)),
            pl.BlockSpec((c, 64), lambda b, i: (0, 0)),
            pl.BlockSpec((1, 64), lambda b, i: (0, 0)),
        ],
        out_specs=[
            pl.BlockSpec((1, blk, k), lambda b, i: (b, i, 0)),
            pl.BlockSpec((1, blk, 64), lambda b, i: (b, i, 0)),
            pl.BlockSpec((1, blk, 64), lambda b, i: (b, i, 0)),
        ],
        out_shape=[
            jax.ShapeDtypeStruct((bsz, n, k), jnp.int32),
            jax.ShapeDtypeStruct((bsz, n, 64), jnp.float32),
            jax.ShapeDtypeStruct((bsz, n, 64), jnp.float32),
        ],
    )(rows, cols, a, d, bias)


def _sc_gather(table, idx):
    info = plsc.get_sparse_core_info()
    nw = info.num_cores * info.num_subcores
    m, dch = idx.shape[0], table.shape[1]
    per_w = m // nw
    ch = 1024
    nch = per_w // ch
    mesh = plsc.VectorSubcoreMesh(core_axis_name="c", subcore_axis_name="s")

    @functools.partial(
        pl.kernel, mesh=mesh,
        compiler_params=pltpu.CompilerParams(use_tc_tiling_on_sc=False),
        out_type=jax.ShapeDtypeStruct((m, dch), jnp.float32),
        scratch_types=[
            pltpu.VMEM((ch,), jnp.int32),
            pltpu.VMEM((ch, dch), jnp.float32),
            pltpu.SemaphoreType.DMA,
        ],
    )
    def gk(table_hbm, idx_hbm, out_hbm, idx_v, rows_v, sem):
        wid = lax.axis_index("s") * info.num_cores + lax.axis_index("c")

        def body(i, carry):
            base = wid * per_w + i * ch
            pltpu.sync_copy(idx_hbm.at[pl.ds(base, ch)], idx_v)
            pltpu.async_copy(table_hbm.at[idx_v], rows_v, sem).wait()
            pltpu.sync_copy(rows_v, out_hbm.at[pl.ds(base, ch)])
            return carry

        lax.fori_loop(0, nch, body, 0)

    return gk(table, idx)


def _shift_sum(zs, k):
    acc = zs[0]
    p = zs[0].shape[0]
    for dd in range(1, len(zs)):
        z = zs[dd]
        shifted = jnp.concatenate(
            [z[:, dd:, :], jnp.zeros((p, dd, 64), jnp.float32)], axis=1)
        acc = acc + shifted
    return acc


def _block_body(g_ref, h_ref, w2_ref, b2_ref, w3_ref, b3_ref, w4_ref, b4_ref,
                out_ref, *, p, k):
    g = g_ref[0].reshape(p, k, 64)
    h = h_ref[0]
    y = jnp.maximum(g + h[:, None, :], 0.0)
    for w_ref, b_ref, taps in ((w2_ref, b2_ref, 2),
                               (w3_ref, b3_ref, 4),
                               (w4_ref, b4_ref, 8)):
        flat = y.reshape(p * k, 64)
        zs = [jnp.dot(flat, w_ref[dd], preferred_element_type=jnp.float32)
                 .reshape(p, k, 64) for dd in range(taps)]
        y = jnp.maximum(_shift_sum(zs, k) + b_ref[...][None, :, :], 0.0)
    out_ref[0] = jnp.max(y[:, : k - 11, :], axis=1)


def _block_call(gath, h, w2, b2, w3, b3, w4, b4, *, k, p):
    bsz, n, _ = h.shape
    grid = (bsz, n // p)
    return pl.pallas_call(
        functools.partial(_block_body, p=p, k=k),
        grid=grid,
        in_specs=[
            pl.BlockSpec((1, p * k, 64), lambda b, i: (b, i, 0)),
            pl.BlockSpec((1, p, 64), lambda b, i: (b, i, 0)),
            pl.BlockSpec((2, 64, 64), lambda b, i: (0, 0, 0)),
            pl.BlockSpec((1, 64), lambda b, i: (0, 0)),
            pl.BlockSpec((4, 64, 64), lambda b, i: (0, 0, 0)),
            pl.BlockSpec((1, 64), lambda b, i: (0, 0)),
            pl.BlockSpec((8, 64, 64), lambda b, i: (0, 0, 0)),
            pl.BlockSpec((1, 64), lambda b, i: (0, 0)),
        ],
        out_specs=pl.BlockSpec((1, p, 64), lambda b, i: (b, i, 0)),
        out_shape=jax.ShapeDtypeStruct((bsz, n, 64), jnp.float32),
    )(gath, h, w2, b2, w3, b3, w4, b4)


def _head_body(x1_ref, x2_ref, p1a_ref, p1b_ref, q1_ref, p2_ref, q2_ref,
               p3_ref, q3_ref, out_ref, *, n):
    z = jnp.maximum(
        jnp.dot(x1_ref[0], p1a_ref[...], preferred_element_type=jnp.float32)
        + jnp.dot(x2_ref[0], p1b_ref[...], preferred_element_type=jnp.float32)
        + q1_ref[...], 0.0)
    z = jnp.maximum(
        jnp.dot(z, p2_ref[...], preferred_element_type=jnp.float32)
        + q2_ref[...], 0.0)
    z = jnp.maximum(
        jnp.dot(z, p3_ref[...], preferred_element_type=jnp.float32)
        + q3_ref[...], 0.0)
    out_ref[0] = jnp.sum(z, axis=0, keepdims=True) * (1.0 / n)


def _head_call(x1, x2, p1a, p1b, q1, p2, q2, p3, q3):
    bsz, n, _ = x1.shape
    no = p3.shape[1]
    return pl.pallas_call(
        functools.partial(_head_body, n=n),
        grid=(bsz,),
        in_specs=[
            pl.BlockSpec((1, n, 64), lambda b: (b, 0, 0)),
            pl.BlockSpec((1, n, 64), lambda b: (b, 0, 0)),
            pl.BlockSpec((64, 64), lambda b: (0, 0)),
            pl.BlockSpec((64, 64), lambda b: (0, 0)),
            pl.BlockSpec((1, 64), lambda b: (0, 0)),
            pl.BlockSpec((64, 256), lambda b: (0, 0)),
            pl.BlockSpec((1, 256), lambda b: (0, 0)),
            pl.BlockSpec((256, no), lambda b: (0, 0)),
            pl.BlockSpec((1, no), lambda b: (0, 0)),
        ],
        out_specs=pl.BlockSpec((1, 1, no), lambda b: (b, 0, 0)),
        out_shape=jax.ShapeDtypeStruct((bsz, 1, no), jnp.float32),
    )(x1, x2, p1a, p1b, q1, p2, q2, p3, q3)


def _fold_block(p, pref, cin):
    s = 1.0 / math.sqrt(1.0 + _EPS)
    s0 = p[pref + "_g0"] * s
    t0 = p[pref + "_b0"]
    w1 = p[pref + "_w1"][:, :, 0, 0]
    s1 = p[pref + "_g1"] * s
    w1eff = w1 * s1[:, None] * s0[None, :]
    bias1 = s1 * (w1 @ t0) + p[pref + "_b1"]
    half = cin // 2
    a = jnp.transpose(w1eff[:, :half])
    d = jnp.transpose(w1eff[:, half:] - w1eff[:, :half])
    ws, bs = [], []
    for i in (2, 3, 4):
        wi = p[pref + "_w%d" % i][:, :, 0, :]
        si = p[pref + "_g%d" % i] * s
        ws.append(jnp.transpose(wi * si[:, None, None], (2, 1, 0)))
        bs.append(p[pref + "_b%d" % i].reshape(1, 64))
    return a, d, bias1.reshape(1, 64), ws, bs


def _fold_head(p, nm):
    s = 1.0 / math.sqrt(1.0 + _EPS)
    w = p["wc" + nm][:, :, 0]
    sc = p["gc" + nm] * s
    wt = jnp.transpose(w * sc[:, None])
    b = (p["bc" + nm] * sc + p["bec" + nm]).reshape(1, -1)
    return wt, b


def kernel(x0, params):
    bsz, _, n = x0.shape
    blk = 512
    p_tile = 256

    a1, d1, bias1, ws1, bs1 = _fold_block(params, "c1", 6)
    a2, d2, bias2, ws2, bs2 = _fold_block(params, "c2", 128)
    p1, q1 = _fold_head(params, "1")
    p2, q2 = _fold_head(params, "2")
    p3, q3 = _fold_head(params, "3")

    xt0 = jnp.transpose(x0, (0, 2, 1))
    idx1, g1, h1 = _knn_call(xt0, x0, a1, d1, bias1, k=_K, blk=blk)
    gath1 = _sc_gather(g1.reshape(bsz * n, 64), idx1.reshape(-1))
    x1 = _block_call(gath1.reshape(bsz, n * _K, 64), h1,
                     ws1[0], bs1[0], ws1[1], bs1[1], ws1[2], bs1[2],
                     k=_K, p=p_tile)

    x1t = jnp.transpose(x1, (0, 2, 1))
    idx2, g2, h2 = _knn_call(x1, x1t, a2, d2, bias2, k=_K, blk=blk)
    gath2 = _sc_gather(g2.reshape(bsz * n, 64), idx2.reshape(-1))
    x2 = _block_call(gath2.reshape(bsz, n * _K, 64), h2,
                     ws2[0], bs2[0], ws2[1], bs2[1], ws2[2], bs2[2],
                     k=_K, p=p_tile)

    out = _head_call(x1, x2, p1[:64], p1[64:], q1, p2, q2, p3, q3)
    return out.reshape(bsz, p3.shape[1])
